# baseline (speedup 1.0000x reference)
import jax
import jax.numpy as jnp
from jax.experimental import pallas as pl
from jax.experimental.pallas import tpu as pltpu

_N, _F, _T = 32, 360, 2048
_TT = 2048


_FC = 8


def _ce_body(out_ref, oh_ref, acc_ref):
    zeros = jnp.zeros((_FC, _TT), jnp.float32)

    def body(k, carry):
        accp, accl, accm = carry
        x = out_ref[0, pl.ds(k * _FC, _FC), :]
        oh = oh_ref[0, pl.ds(k * _FC, _FC), :]
        l = jnp.log(x + 1e-20)
        return (accp + oh * l,
                accl + l,
                jnp.maximum(accm, jnp.abs(oh)))

    accp, accl, accm = jax.lax.fori_loop(
        0, _F // _FC, body, (zeros, zeros, zeros))

    colsum_l = jnp.sum(accl, axis=0)
    zero_col = jnp.max(accm, axis=0) == 0.0
    s_prod = jnp.sum(accp)
    corr = jnp.sum(jnp.where(zero_col, colsum_l, 0.0))
    acc_ref[0, 0, 0] = s_prod + corr * (1.0 / _F)


def kernel(output, one_hot):
    out = jnp.reshape(output, (_N, _F, _T))
    acc = pl.pallas_call(
        _ce_body,
        grid=(_N,),
        in_specs=[
            pl.BlockSpec((1, _F, _TT), lambda i: (i, 0, 0)),
            pl.BlockSpec((1, _F, _TT), lambda i: (i, 0, 0)),
        ],
        out_specs=pl.BlockSpec((1, 1, 1), lambda i: (i, 0, 0),
                               memory_space=pltpu.SMEM),
        out_shape=jax.ShapeDtypeStruct((_N, 1, 1), jnp.float32),
        compiler_params=pltpu.CompilerParams(
            dimension_semantics=("parallel",),
        ),
    )(out, one_hot)
    return -jnp.sum(acc) / (_T * _N)


# speedup vs baseline: 1.1005x; 1.1005x over previous
import jax
import jax.numpy as jnp
from jax.experimental import pallas as pl
from jax.experimental.pallas import tpu as pltpu

_N, _F, _T = 32, 360, 2048
_TT = 2048


def _ce_body(out_ref, oh_ref, acc_ref):
    x = out_ref[0]
    oh = oh_ref[0]
    l = jnp.log(x + 1e-20)

    s_prod = jnp.sum(oh * l)
    colsum_l = jnp.sum(l, axis=0)
    zero_col = jnp.max(jnp.abs(oh), axis=0) == 0.0
    corr = jnp.sum(jnp.where(zero_col, colsum_l, 0.0))
    acc_ref[0, 0, 0] = s_prod + corr * (1.0 / _F)


def kernel(output, one_hot):
    out = jnp.reshape(output, (_N, _F, _T))
    acc = pl.pallas_call(
        _ce_body,
        grid=(_N,),
        in_specs=[
            pl.BlockSpec((1, _F, _TT), lambda i: (i, 0, 0)),
            pl.BlockSpec((1, _F, _TT), lambda i: (i, 0, 0)),
        ],
        out_specs=pl.BlockSpec((1, 1, 1), lambda i: (i, 0, 0),
                               memory_space=pltpu.SMEM),
        out_shape=jax.ShapeDtypeStruct((_N, 1, 1), jnp.float32),
        compiler_params=pltpu.CompilerParams(
            dimension_semantics=("parallel",),
        ),
    )(out, one_hot)
    return -jnp.sum(acc) / (_T * _N)


# speedup vs baseline: 1.2357x; 1.1228x over previous
import jax
import jax.numpy as jnp
from jax.experimental import pallas as pl
from jax.experimental.pallas import tpu as pltpu

_N, _F, _T = 32, 360, 2048
_TT = 2048


_NB = 2


def _ce_body(out_ref, oh_ref, acc_ref):
    x = out_ref[...]
    oh = oh_ref[...]
    l = jnp.log(x + 1e-20)

    s_prod = jnp.sum(oh * l)
    colsum_l = jnp.sum(l, axis=1)
    zero_col = jnp.max(jnp.abs(oh), axis=1) == 0.0
    corr = jnp.sum(jnp.where(zero_col, colsum_l, 0.0))
    acc_ref[0, 0, 0] = s_prod + corr * (1.0 / _F)


def kernel(output, one_hot):
    out = jnp.reshape(output, (_N, _F, _T))
    acc = pl.pallas_call(
        _ce_body,
        grid=(_N // _NB,),
        in_specs=[
            pl.BlockSpec((_NB, _F, _TT), lambda i: (i, 0, 0)),
            pl.BlockSpec((_NB, _F, _TT), lambda i: (i, 0, 0)),
        ],
        out_specs=pl.BlockSpec((1, 1, 1), lambda i: (i, 0, 0),
                               memory_space=pltpu.SMEM),
        out_shape=jax.ShapeDtypeStruct((_N // _NB, 1, 1), jnp.float32),
        compiler_params=pltpu.CompilerParams(
            dimension_semantics=("parallel",),
        ),
    )(out, one_hot)
    return -jnp.sum(acc) / (_T * _N)
